# baseline (speedup 1.0000x reference)
import functools

import jax
import jax.numpy as jnp
from jax import lax
from jax.experimental import pallas as pl
from jax.experimental.pallas import tpu as pltpu
from jax.experimental.pallas import tpu_sc as plsc

N_NODES = 10000
D = 128
E = 320000
NC = 2
NS = 16
NW = NC * NS
C = 128
K = 81
EPT = K * C
E_PAD = NW * EPT
N_PAD = 10112
RPT = N_PAD // NS
NBUF = 3

_mesh = plsc.VectorSubcoreMesh(core_axis_name="c", subcore_axis_name="s")


@functools.partial(
    pl.kernel,
    out_type=jax.ShapeDtypeStruct((NC, N_PAD, D), jnp.float32),
    mesh=_mesh,
    scratch_types=[
        pltpu.VMEM((NBUF, C), jnp.int32),
        pltpu.VMEM((NBUF, C), jnp.int32),
        pltpu.VMEM((NBUF, C, D), jnp.float32),
        pltpu.VMEM_SHARED((N_PAD, D), jnp.float32),
        [pltpu.SemaphoreType.DMA] * NBUF,
        [pltpu.SemaphoreType.DMA] * NBUF,
        [pltpu.SemaphoreType.DMA] * NBUF,
        [pltpu.SemaphoreType.DMA] * NBUF,
    ],
)
def _segsum_sc(src_hbm, dst_hbm, x_hbm, out_hbm, src_r, dst_r, rows_v, acc_sh,
               gsems, dsems, ssems, xsems):
    cid = lax.axis_index("c")
    sid = lax.axis_index("s")
    wid = sid * NC + cid

    zeros16 = jnp.zeros((16,), jnp.float32)

    @pl.loop(0, C)
    def _zero_rows(r):
        for j in range(D // 16):
            rows_v[0, r, pl.ds(j * 16, 16)] = zeros16

    base = sid * RPT
    for t in range(RPT // C):
        pltpu.sync_copy(rows_v.at[0], acc_sh.at[pl.ds(base + t * C, C)])
    if RPT % C:
        pltpu.sync_copy(rows_v.at[0, pl.ds(0, RPT % C)],
                        acc_sh.at[pl.ds(base + (RPT // C) * C, RPT % C)])

    plsc.subcore_barrier()

    def _start_src(chunk, b):
        pltpu.async_copy(src_hbm.at[wid, chunk], src_r.at[b], xsems[b])

    def _wait_src(b):
        pltpu.make_async_copy(src_hbm.at[wid, 0], src_r.at[b], xsems[b]).wait()

    def _start_dst(chunk, b):
        pltpu.async_copy(dst_hbm.at[wid, chunk], dst_r.at[b], dsems[b])

    def _wait_dst(b):
        pltpu.make_async_copy(dst_hbm.at[wid, 0], dst_r.at[b], dsems[b]).wait()

    def _start_gather(b):
        pltpu.async_copy(x_hbm.at[src_r.at[b]], rows_v.at[b], gsems[b])

    def _wait_gather(b):
        pltpu.make_async_copy(x_hbm.at[src_r.at[0]], rows_v.at[b], gsems[b]).wait()

    def _start_scatter(b):
        pltpu.async_copy(rows_v.at[b], acc_sh.at[dst_r.at[b]], ssems[b], add=True)

    def _wait_scatter(b):
        pltpu.make_async_copy(rows_v.at[b], acc_sh.at[dst_r.at[0]],
                              ssems[b]).wait()

    for b in range(NBUF):
        _start_src(b, b)
        _start_dst(b, b)
    for b in range(NBUF):
        _wait_src(b)
        _start_gather(b)

    def _visit(k, b, bprev, guard):
        _wait_gather(b)
        _wait_dst(b)
        if guard:
            nxt = k + NBUF

            @pl.when(nxt < K)
            def _():
                _start_src(nxt, b)
        else:
            _start_src(k + NBUF, b)
        _start_scatter(b)
        if bprev is not None:
            _wait_scatter(bprev)
            j = k + NBUF - 1
            if guard:
                @pl.when(j < K)
                def _():
                    _wait_src(bprev)
                    _start_gather(bprev)
                    _start_dst(j, bprev)
            else:
                _wait_src(bprev)
                _start_gather(bprev)
                _start_dst(j, bprev)

    _visit(0, 0, None, False)
    for k in range(1, NBUF):
        _visit(k, k % NBUF, (k - 1) % NBUF, False)

    @pl.loop(NBUF, K, step=NBUF)
    def _edge_group(k0):
        for b in range(NBUF):
            _visit(k0 + b, b, (b - 1) % NBUF, True)

    _wait_scatter((K - 1) % NBUF)

    plsc.subcore_barrier()

    pltpu.sync_copy(acc_sh.at[pl.ds(base, RPT)], out_hbm.at[cid, pl.ds(base, RPT)])


def _mlp_body(x_ref, p_ref, w1t_ref, b1_ref, g_ref, be_ref, w2t_ref, b2_ref,
              eps_ref, o_ref):
    h = (x_ref[...] * (1.0 + eps_ref[0, 0])
         + p_ref[0, :N_NODES, :] + p_ref[1, :N_NODES, :])
    z = jnp.dot(h, w1t_ref[...], preferred_element_type=jnp.float32) + b1_ref[...]
    mean = jnp.mean(z, axis=0, keepdims=True)
    zc = z - mean
    var = jnp.mean(zc * zc, axis=0, keepdims=True)
    y = zc * lax.rsqrt(var + 1e-5) * g_ref[...] + be_ref[...]
    y = jnp.maximum(y, 0.0)
    o_ref[...] = jnp.dot(y, w2t_ref[...], preferred_element_type=jnp.float32) + b2_ref[...]


def _mlp_tc(x, partials, W1t, b1, gamma, beta, W2t, b2, eps):
    return pl.pallas_call(
        _mlp_body,
        out_shape=jax.ShapeDtypeStruct((N_NODES, D), jnp.float32),
    )(x, partials, W1t, b1.reshape(1, D), gamma.reshape(1, D),
      beta.reshape(1, D), W2t, b2.reshape(1, D), eps.reshape(1, 1))


def kernel(x, edge_index, W1, b1, gamma, beta, W2, b2, eps):
    ppt = EPT - E // NW
    src = jnp.concatenate(
        [edge_index[0].astype(jnp.int32).reshape(NW, E // NW),
         jnp.zeros((NW, ppt), jnp.int32)], axis=1).reshape(NW, K, C)
    junk = (N_NODES
            + (jnp.arange(NW * ppt, dtype=jnp.int32) % (N_PAD - N_NODES))
            ).reshape(NW, ppt)
    dst = jnp.concatenate(
        [edge_index[1].astype(jnp.int32).reshape(NW, E // NW), junk],
        axis=1).reshape(NW, K, C)
    partials = _segsum_sc(src, dst, x)
    return _mlp_tc(x, partials, W1.T, b1, gamma, beta, W2.T, b2, eps)


# speedup vs baseline: 2.4777x; 2.4777x over previous
import functools

import jax
import jax.numpy as jnp
from jax import lax
from jax.experimental import pallas as pl
from jax.experimental.pallas import tpu as pltpu
from jax.experimental.pallas import tpu_sc as plsc

N_NODES = 10000
D = 128
E = 320000
NC = 2
NS = 16
NW = NC * NS
C = 120
K = 84
EPT = K * C
E_PAD = NW * EPT
N_PAD = 10112
RPT = N_PAD // NS
NBUF = 3

_mesh = plsc.VectorSubcoreMesh(core_axis_name="c", subcore_axis_name="s")


@functools.partial(
    pl.kernel,
    out_type=jax.ShapeDtypeStruct((NC, N_PAD, D), jnp.float32),
    mesh=_mesh,
    scratch_types=[
        pltpu.VMEM((NBUF, C), jnp.int32),
        pltpu.VMEM((NBUF, C), jnp.int32),
        pltpu.VMEM((NBUF, C, D), jnp.float32),
        pltpu.VMEM_SHARED((N_PAD, D), jnp.float32),
        [pltpu.SemaphoreType.DMA] * NBUF,
        [pltpu.SemaphoreType.DMA] * NBUF,
        [pltpu.SemaphoreType.DMA] * NBUF,
        [pltpu.SemaphoreType.DMA] * NBUF,
    ],
)
def _segsum_sc(src_hbm, dst_hbm, x_hbm, out_hbm, src_r, dst_r, rows_v, acc_sh,
               gsems, dsems, ssems, xsems):
    cid = lax.axis_index("c")
    sid = lax.axis_index("s")
    wid = sid * NC + cid

    zeros16 = jnp.zeros((16,), jnp.float32)

    @pl.loop(0, C)
    def _zero_rows(r):
        for j in range(D // 16):
            rows_v[0, r, pl.ds(j * 16, 16)] = zeros16

    base = sid * RPT
    for t in range(RPT // C):
        pltpu.sync_copy(rows_v.at[0], acc_sh.at[pl.ds(base + t * C, C)])
    if RPT % C:
        pltpu.sync_copy(rows_v.at[0, pl.ds(0, RPT % C)],
                        acc_sh.at[pl.ds(base + (RPT // C) * C, RPT % C)])

    plsc.subcore_barrier()

    def _start_src(chunk, b):
        pltpu.async_copy(src_hbm.at[wid, chunk], src_r.at[b], xsems[b])

    def _wait_src(b):
        pltpu.make_async_copy(src_hbm.at[wid, 0], src_r.at[b], xsems[b]).wait()

    def _start_dst(chunk, b):
        pltpu.async_copy(dst_hbm.at[wid, chunk], dst_r.at[b], dsems[b])

    def _wait_dst(b):
        pltpu.make_async_copy(dst_hbm.at[wid, 0], dst_r.at[b], dsems[b]).wait()

    def _start_gather(b):
        pltpu.async_copy(x_hbm.at[src_r.at[b]], rows_v.at[b], gsems[b])

    def _wait_gather(b):
        pltpu.make_async_copy(x_hbm.at[src_r.at[0]], rows_v.at[b], gsems[b]).wait()

    def _start_scatter(b):
        pltpu.async_copy(rows_v.at[b], acc_sh.at[dst_r.at[b]], ssems[b], add=True)

    def _wait_scatter(b):
        pltpu.make_async_copy(rows_v.at[b], acc_sh.at[dst_r.at[0]],
                              ssems[b]).wait()

    for b in range(NBUF):
        _start_src(b, b)
        _start_dst(b, b)
    for b in range(NBUF):
        _wait_src(b)
        _start_gather(b)

    def _visit(k, b, bprev, guard):
        _wait_gather(b)
        _wait_dst(b)
        if guard:
            nxt = k + NBUF

            @pl.when(nxt < K)
            def _():
                _start_src(nxt, b)
        else:
            _start_src(k + NBUF, b)
        _start_scatter(b)
        if bprev is not None:
            _wait_scatter(bprev)
            j = k + NBUF - 1
            if guard:
                @pl.when(j < K)
                def _():
                    _wait_src(bprev)
                    _start_gather(bprev)
                    _start_dst(j, bprev)
            else:
                _wait_src(bprev)
                _start_gather(bprev)
                _start_dst(j, bprev)

    _visit(0, 0, None, False)
    for k in range(1, NBUF):
        _visit(k, k % NBUF, (k - 1) % NBUF, False)

    @pl.loop(NBUF, K, step=NBUF)
    def _edge_group(k0):
        for b in range(NBUF):
            _visit(k0 + b, b, (b - 1) % NBUF, True)

    _wait_scatter((K - 1) % NBUF)

    plsc.subcore_barrier()

    pltpu.sync_copy(acc_sh.at[pl.ds(base, RPT)], out_hbm.at[cid, pl.ds(base, RPT)])


def _mlp_body(x_ref, p_ref, w1t_ref, b1_ref, g_ref, be_ref, w2t_ref, b2_ref,
              eps_ref, o_ref):
    h = (x_ref[...] * (1.0 + eps_ref[0, 0])
         + p_ref[0, :N_NODES, :] + p_ref[1, :N_NODES, :])
    z = jnp.dot(h, w1t_ref[...], preferred_element_type=jnp.float32) + b1_ref[...]
    mean = jnp.mean(z, axis=0, keepdims=True)
    zc = z - mean
    var = jnp.mean(zc * zc, axis=0, keepdims=True)
    y = zc * lax.rsqrt(var + 1e-5) * g_ref[...] + be_ref[...]
    y = jnp.maximum(y, 0.0)
    o_ref[...] = jnp.dot(y, w2t_ref[...], preferred_element_type=jnp.float32) + b2_ref[...]


def _mlp_tc(x, partials, W1t, b1, gamma, beta, W2t, b2, eps):
    return pl.pallas_call(
        _mlp_body,
        out_shape=jax.ShapeDtypeStruct((N_NODES, D), jnp.float32),
    )(x, partials, W1t, b1.reshape(1, D), gamma.reshape(1, D),
      beta.reshape(1, D), W2t, b2.reshape(1, D), eps.reshape(1, 1))


def kernel(x, edge_index, W1, b1, gamma, beta, W2, b2, eps):
    ppt = EPT - E // NW
    src = jnp.concatenate(
        [edge_index[0].astype(jnp.int32).reshape(NW, E // NW),
         jnp.zeros((NW, ppt), jnp.int32)], axis=1).reshape(NW, K, C)
    junk = (N_NODES
            + (jnp.arange(NW * ppt, dtype=jnp.int32) % (N_PAD - N_NODES))
            ).reshape(NW, ppt)
    dst = jnp.concatenate(
        [edge_index[1].astype(jnp.int32).reshape(NW, E // NW), junk],
        axis=1).reshape(NW, K, C)
    partials = _segsum_sc(src, dst, x)
    return _mlp_tc(x, partials, W1.T, b1, gamma, beta, W2.T, b2, eps)
